# Initial kernel scaffold; baseline (speedup 1.0000x reference)
import jax
import jax.numpy as jnp
from jax.experimental import pallas as pl


def kernel(nodes, neigh_pos, neigh_neg, features, W_bal, W_unbal):
    raise NotImplementedError("write your pallas kernel here")


# speedup vs baseline: 1.4605x; 1.4605x over previous
import functools

import jax
import jax.numpy as jnp
from jax import lax
from jax.experimental import pallas as pl
from jax.experimental.pallas import tpu as pltpu
from jax.experimental.pallas import tpu_sc as plsc

N_NODES = 100000
D = 128
B = 20000
S = 10

NW = 32
BP = 20480
BPW = BP // NW
C = 8
NCH = BPW // C
NV = D // 16


def _sc_body(nodes_hbm, posf_hbm, negf_hbm, feat_hbm,
             self_out, aggp_out, aggn_out,
             idxs_v, idxp_v, idxn_v, selfr_v, posr_v, negr_v,
             aggp_v, aggn_v, sem):
    wid = lax.axis_index("s") * 2 + lax.axis_index("c")
    wbase = wid * BPW

    def chunk(g, carry):
        row0 = pl.multiple_of(wbase + g * C, 8)
        irow0 = pl.multiple_of(row0 * S, 8)
        pltpu.sync_copy(nodes_hbm.at[pl.ds(row0, C)], idxs_v)
        pltpu.sync_copy(posf_hbm.at[pl.ds(irow0, C * S)], idxp_v)
        pltpu.sync_copy(negf_hbm.at[pl.ds(irow0, C * S)], idxn_v)
        cps = pltpu.async_copy(feat_hbm.at[idxs_v], selfr_v, sem)
        cpp = pltpu.async_copy(feat_hbm.at[idxp_v], posr_v, sem)
        cpn = pltpu.async_copy(feat_hbm.at[idxn_v], negr_v, sem)
        cps.wait()
        cpp.wait()
        cpn.wait()
        for i in range(C):
            for v in range(NV):
                sl = pl.ds(v * 16, 16)
                accp = posr_v[i * S, sl]
                accn = negr_v[i * S, sl]
                for s in range(1, S):
                    accp = accp + posr_v[i * S + s, sl]
                    accn = accn + negr_v[i * S + s, sl]
                aggp_v[i, sl] = accp * (1.0 / S)
                aggn_v[i, sl] = accn * (1.0 / S)
        pltpu.sync_copy(selfr_v, self_out.at[pl.ds(row0, C)])
        pltpu.sync_copy(aggp_v, aggp_out.at[pl.ds(row0, C)])
        pltpu.sync_copy(aggn_v, aggn_out.at[pl.ds(row0, C)])
        return carry

    lax.fori_loop(0, NCH, chunk, 0)


_sc_gather_agg = functools.partial(
    pl.kernel,
    out_type=[jax.ShapeDtypeStruct((BP, D), jnp.float32)] * 3,
    mesh=plsc.VectorSubcoreMesh(core_axis_name="c", subcore_axis_name="s"),
    scratch_types=[
        pltpu.VMEM((C,), jnp.int32),
        pltpu.VMEM((C * S,), jnp.int32),
        pltpu.VMEM((C * S,), jnp.int32),
        pltpu.VMEM((C, D), jnp.float32),
        pltpu.VMEM((C * S, D), jnp.float32),
        pltpu.VMEM((C * S, D), jnp.float32),
        pltpu.VMEM((C, D), jnp.float32),
        pltpu.VMEM((C, D), jnp.float32),
        pltpu.SemaphoreType.DMA,
    ],
)(_sc_body)


def _tc_body(self_ref, aggp_ref, aggn_ref, wb_ref, wu_ref, ob_ref, ou_ref):
    s = self_ref[...]
    wb = wb_ref[...]
    wu = wu_ref[...]
    bal = (jnp.dot(s, wb[:D], preferred_element_type=jnp.float32)
           + jnp.dot(aggp_ref[...], wb[D:], preferred_element_type=jnp.float32))
    unbal = (jnp.dot(s, wu[:D], preferred_element_type=jnp.float32)
             + jnp.dot(aggn_ref[...], wu[D:], preferred_element_type=jnp.float32))
    ob_ref[...] = jnp.tanh(bal)
    ou_ref[...] = jnp.tanh(unbal)


_TC_BS = 2048


def _tc_encode(selff, aggp, aggn, W_bal, W_unbal):
    grid = BP // _TC_BS
    row_spec = pl.BlockSpec((_TC_BS, D), lambda i: (i, 0))
    w_spec = pl.BlockSpec((2 * D, D), lambda i: (0, 0))
    return pl.pallas_call(
        _tc_body,
        grid=(grid,),
        in_specs=[row_spec, row_spec, row_spec, w_spec, w_spec],
        out_specs=[row_spec, row_spec],
        out_shape=[jax.ShapeDtypeStruct((BP, D), jnp.float32)] * 2,
    )(selff, aggp, aggn, W_bal, W_unbal)


def kernel(nodes, neigh_pos, neigh_neg, features, W_bal, W_unbal):
    nodes_p = jnp.pad(nodes, (0, BP - B))
    posf = jnp.pad(neigh_pos.reshape(-1), (0, (BP - B) * S))
    negf = jnp.pad(neigh_neg.reshape(-1), (0, (BP - B) * S))
    selff, aggp, aggn = _sc_gather_agg(nodes_p, posf, negf, features)
    ob, ou = _tc_encode(selff, aggp, aggn, W_bal, W_unbal)
    return ob[:B], ou[:B]


# speedup vs baseline: 2.0433x; 1.3991x over previous
import functools

import jax
import jax.numpy as jnp
from jax import lax
from jax.experimental import pallas as pl
from jax.experimental.pallas import tpu as pltpu
from jax.experimental.pallas import tpu_sc as plsc

N_NODES = 100000
D = 128
B = 20000
S = 10

NW = 32
BP = 20480
BPW = BP // NW
C = 8
NCH = BPW // C
NV = D // 16


def _sc_body(nodes_hbm, posf_hbm, negf_hbm, feat_hbm,
             self_out, aggp_out, aggn_out,
             idxs_v, idxp_v, idxn_v,
             selfr0, posr0, negr0, selfr1, posr1, negr1,
             aggp0, aggn0, aggp1, aggn1,
             semg0, semg1, semo0, semo1):
    wid = lax.axis_index("s") * 2 + lax.axis_index("c")
    wbase = wid * BPW

    pltpu.sync_copy(nodes_hbm.at[pl.ds(pl.multiple_of(wbase, 8), BPW)], idxs_v)
    pltpu.sync_copy(posf_hbm.at[pl.ds(pl.multiple_of(wbase * S, 8), BPW * S)],
                    idxp_v)
    pltpu.sync_copy(negf_hbm.at[pl.ds(pl.multiple_of(wbase * S, 8), BPW * S)],
                    idxn_v)

    bufs = ((selfr0, posr0, negr0, aggp0, aggn0, semg0, semo0),
            (selfr1, posr1, negr1, aggp1, aggn1, semg1, semo1))

    def issue_g(g, b):
        selfr, posr, negr, _, _, semg, _ = bufs[b]
        off = pl.multiple_of(g * C, 8)
        ioff = pl.multiple_of(g * C * S, 8)
        pltpu.async_copy(feat_hbm.at[idxs_v.at[pl.ds(off, C)]], selfr, semg)
        pltpu.async_copy(feat_hbm.at[idxp_v.at[pl.ds(ioff, C * S)]], posr, semg)
        pltpu.async_copy(feat_hbm.at[idxn_v.at[pl.ds(ioff, C * S)]], negr, semg)

    def wait_g(b):
        selfr, posr, negr, _, _, semg, _ = bufs[b]
        pltpu.make_async_copy(feat_hbm.at[idxs_v.at[pl.ds(0, C)]],
                              selfr, semg).wait()
        pltpu.make_async_copy(feat_hbm.at[idxp_v.at[pl.ds(0, C * S)]],
                              posr, semg).wait()
        pltpu.make_async_copy(feat_hbm.at[idxn_v.at[pl.ds(0, C * S)]],
                              negr, semg).wait()

    def wait_out(b):
        _, _, _, aggp, aggn, _, semo = bufs[b]
        row = pl.ds(0, C)
        pltpu.make_async_copy(aggp, aggp_out.at[row], semo).wait()
        pltpu.make_async_copy(aggn, aggn_out.at[row], semo).wait()

    def step(g, b, p):
        selfr, posr, negr, aggp, aggn, _, semo = bufs[b]
        row0 = pl.multiple_of(wbase + g * C, 8)
        wait_g(b)
        pltpu.sync_copy(selfr, self_out.at[pl.ds(row0, C)])

        @pl.when(p > 0)
        def _():
            wait_out(b)

        for i in range(C):
            for v in range(NV):
                sl = pl.ds(v * 16, 16)
                accp = posr[i * S, sl]
                accn = negr[i * S, sl]
                for s in range(1, S):
                    accp = accp + posr[i * S + s, sl]
                    accn = accn + negr[i * S + s, sl]
                aggp[i, sl] = accp * (1.0 / S)
                aggn[i, sl] = accn * (1.0 / S)
        pltpu.async_copy(aggp, aggp_out.at[pl.ds(row0, C)], semo)
        pltpu.async_copy(aggn, aggn_out.at[pl.ds(row0, C)], semo)

    P = NCH // 2
    issue_g(0, 0)

    def pair(p, carry):
        issue_g(2 * p + 1, 1)
        step(2 * p, 0, p)

        @pl.when(p < P - 1)
        def _():
            issue_g(2 * p + 2, 0)

        step(2 * p + 1, 1, p)
        return carry

    lax.fori_loop(0, P, pair, 0)
    wait_out(0)
    wait_out(1)


_sc_gather_agg = functools.partial(
    pl.kernel,
    out_type=[jax.ShapeDtypeStruct((BP, D), jnp.float32)] * 3,
    mesh=plsc.VectorSubcoreMesh(core_axis_name="c", subcore_axis_name="s"),
    scratch_types=[
        pltpu.VMEM((BPW,), jnp.int32),
        pltpu.VMEM((BPW * S,), jnp.int32),
        pltpu.VMEM((BPW * S,), jnp.int32),
        pltpu.VMEM((C, D), jnp.float32),
        pltpu.VMEM((C * S, D), jnp.float32),
        pltpu.VMEM((C * S, D), jnp.float32),
        pltpu.VMEM((C, D), jnp.float32),
        pltpu.VMEM((C * S, D), jnp.float32),
        pltpu.VMEM((C * S, D), jnp.float32),
        pltpu.VMEM((C, D), jnp.float32),
        pltpu.VMEM((C, D), jnp.float32),
        pltpu.VMEM((C, D), jnp.float32),
        pltpu.VMEM((C, D), jnp.float32),
        pltpu.SemaphoreType.DMA,
        pltpu.SemaphoreType.DMA,
        pltpu.SemaphoreType.DMA,
        pltpu.SemaphoreType.DMA,
    ],
)(_sc_body)


def _tc_body(self_ref, aggp_ref, aggn_ref, wb_ref, wu_ref, ob_ref, ou_ref):
    s = self_ref[...]
    wb = wb_ref[...]
    wu = wu_ref[...]
    bal = (jnp.dot(s, wb[:D], preferred_element_type=jnp.float32)
           + jnp.dot(aggp_ref[...], wb[D:], preferred_element_type=jnp.float32))
    unbal = (jnp.dot(s, wu[:D], preferred_element_type=jnp.float32)
             + jnp.dot(aggn_ref[...], wu[D:], preferred_element_type=jnp.float32))
    ob_ref[...] = jnp.tanh(bal)
    ou_ref[...] = jnp.tanh(unbal)


_TC_BS = 2048


def _tc_encode(selff, aggp, aggn, W_bal, W_unbal):
    grid = BP // _TC_BS
    row_spec = pl.BlockSpec((_TC_BS, D), lambda i: (i, 0))
    w_spec = pl.BlockSpec((2 * D, D), lambda i: (0, 0))
    return pl.pallas_call(
        _tc_body,
        grid=(grid,),
        in_specs=[row_spec, row_spec, row_spec, w_spec, w_spec],
        out_specs=[row_spec, row_spec],
        out_shape=[jax.ShapeDtypeStruct((BP, D), jnp.float32)] * 2,
    )(selff, aggp, aggn, W_bal, W_unbal)


def kernel(nodes, neigh_pos, neigh_neg, features, W_bal, W_unbal):
    nodes_p = jnp.pad(nodes, (0, BP - B))
    posf = jnp.pad(neigh_pos.reshape(-1), (0, (BP - B) * S))
    negf = jnp.pad(neigh_neg.reshape(-1), (0, (BP - B) * S))
    selff, aggp, aggn = _sc_gather_agg(nodes_p, posf, negf, features)
    ob, ou = _tc_encode(selff, aggp, aggn, W_bal, W_unbal)
    return ob[:B], ou[:B]


# speedup vs baseline: 2.1917x; 1.0726x over previous
import functools

import jax
import jax.numpy as jnp
from jax import lax
from jax.experimental import pallas as pl
from jax.experimental.pallas import tpu as pltpu
from jax.experimental.pallas import tpu_sc as plsc

N_NODES = 100000
D = 128
B = 20000
S = 10

NW = 32
BP = 20480
BPW = BP // NW
C = 8
NCH = BPW // C
NV = D // 16


NBUF = 4


def _sc_body(nodes_hbm, posf_hbm, negf_hbm, feat_hbm,
             self_out, aggp_out, aggn_out,
             idxs_v, idxp_v, idxn_v, bufs, semgs, semos):
    wid = lax.axis_index("s") * 2 + lax.axis_index("c")
    wbase = wid * BPW

    pltpu.sync_copy(nodes_hbm.at[pl.ds(pl.multiple_of(wbase, 8), BPW)], idxs_v)
    pltpu.sync_copy(posf_hbm.at[pl.ds(pl.multiple_of(wbase * S, 8), BPW * S)],
                    idxp_v)
    pltpu.sync_copy(negf_hbm.at[pl.ds(pl.multiple_of(wbase * S, 8), BPW * S)],
                    idxn_v)

    def issue_g(g, b):
        selfr, posr, negr, _, _ = bufs[b]
        off = pl.multiple_of(g * C, 8)
        ioff = pl.multiple_of(g * C * S, 8)
        pltpu.async_copy(feat_hbm.at[idxs_v.at[pl.ds(off, C)]], selfr, semgs[b])
        pltpu.async_copy(feat_hbm.at[idxp_v.at[pl.ds(ioff, C * S)]], posr,
                         semgs[b])
        pltpu.async_copy(feat_hbm.at[idxn_v.at[pl.ds(ioff, C * S)]], negr,
                         semgs[b])

    def wait_g(b):
        selfr, posr, negr, _, _ = bufs[b]
        pltpu.make_async_copy(feat_hbm.at[idxs_v.at[pl.ds(0, C)]],
                              selfr, semgs[b]).wait()
        pltpu.make_async_copy(feat_hbm.at[idxp_v.at[pl.ds(0, C * S)]],
                              posr, semgs[b]).wait()
        pltpu.make_async_copy(feat_hbm.at[idxn_v.at[pl.ds(0, C * S)]],
                              negr, semgs[b]).wait()

    def wait_out(b):
        _, _, _, aggp, aggn = bufs[b]
        row = pl.ds(0, C)
        pltpu.make_async_copy(aggp, aggp_out.at[row], semos[b]).wait()
        pltpu.make_async_copy(aggn, aggn_out.at[row], semos[b]).wait()

    def step(g, b, bnext, p):
        selfr, posr, negr, aggp, aggn = bufs[b]
        row0 = pl.multiple_of(wbase + g * C, 8)
        wait_g(b)
        pltpu.sync_copy(selfr, self_out.at[pl.ds(row0, C)])

        @pl.when(p > 0)
        def _():
            wait_out(b)

        def seed(i, carry):
            row = i * S
            for v in range(NV):
                sl = pl.ds(v * 16, 16)
                accp = posr[row, sl]
                accn = negr[row, sl]
                for s in range(1, S):
                    accp = accp + posr[row + s, sl]
                    accn = accn + negr[row + s, sl]
                aggp[i, sl] = accp * (1.0 / S)
                aggn[i, sl] = accn * (1.0 / S)
            return carry

        lax.fori_loop(0, C, seed, 0)
        pltpu.async_copy(aggp, aggp_out.at[pl.ds(row0, C)], semos[b])
        pltpu.async_copy(aggn, aggn_out.at[pl.ds(row0, C)], semos[b])

        @pl.when(g + NBUF - 1 < NCH)
        def _():
            issue_g(g + NBUF - 1, bnext)

    for j in range(NBUF - 1):
        issue_g(j, j)

    P = NCH // NBUF

    def group(p, carry):
        for j in range(NBUF):
            step(p * NBUF + j, j, (j - 1) % NBUF, p)
        return carry

    lax.fori_loop(0, P, group, 0)
    for b in range(NBUF):
        wait_out(b)


_sc_gather_agg = functools.partial(
    pl.kernel,
    out_type=[jax.ShapeDtypeStruct((BP, D), jnp.float32)] * 3,
    mesh=plsc.VectorSubcoreMesh(core_axis_name="c", subcore_axis_name="s"),
    scratch_types=[
        pltpu.VMEM((BPW,), jnp.int32),
        pltpu.VMEM((BPW * S,), jnp.int32),
        pltpu.VMEM((BPW * S,), jnp.int32),
        tuple(
            (pltpu.VMEM((C, D), jnp.float32),
             pltpu.VMEM((C * S, D), jnp.float32),
             pltpu.VMEM((C * S, D), jnp.float32),
             pltpu.VMEM((C, D), jnp.float32),
             pltpu.VMEM((C, D), jnp.float32))
            for _ in range(NBUF)),
        tuple(pltpu.SemaphoreType.DMA for _ in range(NBUF)),
        tuple(pltpu.SemaphoreType.DMA for _ in range(NBUF)),
    ],
)(_sc_body)


def _tc_body(self_ref, aggp_ref, aggn_ref, wb_ref, wu_ref, ob_ref, ou_ref):
    s = self_ref[...]
    wb = wb_ref[...]
    wu = wu_ref[...]
    bal = (jnp.dot(s, wb[:D], preferred_element_type=jnp.float32)
           + jnp.dot(aggp_ref[...], wb[D:], preferred_element_type=jnp.float32))
    unbal = (jnp.dot(s, wu[:D], preferred_element_type=jnp.float32)
             + jnp.dot(aggn_ref[...], wu[D:], preferred_element_type=jnp.float32))
    ob_ref[...] = jnp.tanh(bal)
    ou_ref[...] = jnp.tanh(unbal)


_TC_BS = 2048


def _tc_encode(selff, aggp, aggn, W_bal, W_unbal):
    grid = BP // _TC_BS
    row_spec = pl.BlockSpec((_TC_BS, D), lambda i: (i, 0))
    w_spec = pl.BlockSpec((2 * D, D), lambda i: (0, 0))
    return pl.pallas_call(
        _tc_body,
        grid=(grid,),
        in_specs=[row_spec, row_spec, row_spec, w_spec, w_spec],
        out_specs=[row_spec, row_spec],
        out_shape=[jax.ShapeDtypeStruct((BP, D), jnp.float32)] * 2,
    )(selff, aggp, aggn, W_bal, W_unbal)


def kernel(nodes, neigh_pos, neigh_neg, features, W_bal, W_unbal):
    nodes_p = jnp.pad(nodes, (0, BP - B))
    posf = jnp.pad(neigh_pos.reshape(-1), (0, (BP - B) * S))
    negf = jnp.pad(neigh_neg.reshape(-1), (0, (BP - B) * S))
    selff, aggp, aggn = _sc_gather_agg(nodes_p, posf, negf, features)
    ob, ou = _tc_encode(selff, aggp, aggn, W_bal, W_unbal)
    return ob[:B], ou[:B]
